# baseline (speedup 1.0000x reference)
import functools

import jax
import jax.numpy as jnp
from jax.experimental import pallas as pl
from jax.experimental.pallas import tpu as pltpu


def _lsh_kernel(x_ref, r_ref, o_ref, *, rounds, L):
    x = x_ref[0]
    r = r_ref[0]
    h = jnp.dot(x, r, preferred_element_type=jnp.float32)
    rows = x.shape[0]
    iota = jax.lax.broadcasted_iota(jnp.int32, (rows, L), 1)
    parts = []
    for k in range(rounds):
        hk = h[:, k * L:(k + 1) * L]
        m = jnp.max(jnp.maximum(hk, -hk), axis=1, keepdims=True)
        idx = jnp.where(hk == m, iota,
                        jnp.where(-hk == m, iota + L, 2 * L))
        parts.append(jnp.min(idx, axis=1, keepdims=True))
    o_ref[0] = jnp.concatenate(parts, axis=1)


def kernel(inp, rand_matrix, n_buckets):
    B, S, D = inp.shape
    _, _, R, L = rand_matrix.shape
    rm = rand_matrix.reshape(B, D, R * L)
    CHUNK = 1024
    grid = (B, S // CHUNK)
    return pl.pallas_call(
        functools.partial(_lsh_kernel, rounds=R, L=L),
        grid=grid,
        in_specs=[
            pl.BlockSpec((1, CHUNK, D), lambda b, s: (b, s, 0)),
            pl.BlockSpec((1, D, R * L), lambda b, s: (b, 0, 0)),
        ],
        out_specs=pl.BlockSpec((1, CHUNK, R), lambda b, s: (b, s, 0)),
        out_shape=jax.ShapeDtypeStruct((B, S, R), jnp.int32),
        compiler_params=pltpu.CompilerParams(
            dimension_semantics=("parallel", "parallel"),
        ),
    )(inp, rm)


# speedup vs baseline: 2.3779x; 2.3779x over previous
import functools

import jax
import jax.numpy as jnp
from jax.experimental import pallas as pl
from jax.experimental.pallas import tpu as pltpu


def _lsh_kernel(x_ref, rt_ref, o_ref, *, rounds, L):
    x = x_ref[0]
    rt = rt_ref[0]
    hT = jax.lax.dot_general(
        rt, x, (((1,), (1,)), ((), ())),
        preferred_element_type=jnp.float32)
    tokens = x.shape[0]
    iota = jax.lax.broadcasted_iota(jnp.int32, (L, tokens), 0)
    parts = []
    for k in range(rounds):
        hk = hT[k * L:(k + 1) * L, :]
        maxp = jnp.max(hk, axis=0, keepdims=True)
        minp = jnp.min(hk, axis=0, keepdims=True)
        m = jnp.maximum(maxp, -minp)
        idx = jnp.where(hk == m, iota,
                        jnp.where(hk == -m, iota + L, 2 * L))
        parts.append(jnp.min(idx, axis=0, keepdims=True))
    o_ref[0] = jnp.concatenate(parts, axis=0)


def kernel(inp, rand_matrix, n_buckets):
    B, S, D = inp.shape
    _, _, R, L = rand_matrix.shape
    rt = rand_matrix.transpose(0, 2, 3, 1).reshape(B, R * L, D)
    CHUNK = 1024
    grid = (B, S // CHUNK)
    out = pl.pallas_call(
        functools.partial(_lsh_kernel, rounds=R, L=L),
        grid=grid,
        in_specs=[
            pl.BlockSpec((1, CHUNK, D), lambda b, s: (b, s, 0)),
            pl.BlockSpec((1, R * L, D), lambda b, s: (b, 0, 0)),
        ],
        out_specs=pl.BlockSpec((1, R, CHUNK), lambda b, s: (b, 0, s)),
        out_shape=jax.ShapeDtypeStruct((B, R, S), jnp.int32),
        compiler_params=pltpu.CompilerParams(
            dimension_semantics=("parallel", "parallel"),
        ),
    )(inp, rt)
    return out.transpose(0, 2, 1)
